# Initial kernel scaffold; baseline (speedup 1.0000x reference)
import functools

import jax
import jax.numpy as jnp
from jax import lax
from jax.experimental import pallas as pl
from jax.experimental.pallas import tpu as pltpu
from jax.experimental.pallas import tpu_sc as plsc

N = 10000
E = 160000
D_IN = 256
D_OUT = 256
H = 128
NP = 10240
NS = 16
STRIPE = NP // NS
CH = 80
EPT = E // NS
NCHUNK = EPT // CH

_MESH = plsc.VectorSubcoreMesh(core_axis_name="c", subcore_axis_name="s")


def _hist_body(row_hbm, ones_hbm, zeros_hbm, table_hbm, ridx_v, ones_v, table_sh):
    c = lax.axis_index("c")
    s = lax.axis_index("s")

    @pl.when(c == 0)
    def _():
        pltpu.sync_copy(zeros_hbm.at[pl.ds(s * STRIPE, STRIPE)],
                        table_sh.at[pl.ds(s * STRIPE, STRIPE)])
        pltpu.sync_copy(ones_hbm, ones_v)
        plsc.subcore_barrier()

        def chunk(g, carry):
            e0 = s * EPT + g * CH
            pltpu.sync_copy(row_hbm.at[pl.ds(e0, CH)], ridx_v)
            pltpu.sync_copy(ones_v, table_sh.at[ridx_v], add=True)
            return carry

        lax.fori_loop(0, NCHUNK, chunk, 0)
        plsc.subcore_barrier()
        pltpu.sync_copy(table_sh.at[pl.ds(s * STRIPE, STRIPE)],
                        table_hbm.at[pl.ds(s * STRIPE, STRIPE)])


_hist = pl.kernel(
    _hist_body,
    out_type=jax.ShapeDtypeStruct((NP, 16), jnp.float32),
    mesh=_MESH,
    scratch_types=[
        pltpu.VMEM((CH,), jnp.int32),
        pltpu.VMEM((CH, 16), jnp.float32),
        pltpu.VMEM_SHARED((NP, 16), jnp.float32),
    ],
)


def _mm_body(x_ref, w_ref, tbl_ref, y0_ref, y1_ref, dis_ref):
    deg = tbl_ref[:, 0:1]
    dis = jnp.where(deg > 0.0, lax.rsqrt(deg), 0.0)
    y = jnp.dot(x_ref[...], w_ref[...], preferred_element_type=jnp.float32) * dis
    y0_ref[...] = y[:, :H]
    y1_ref[...] = y[:, H:]
    dis_ref[...] = dis


_RB = 1000


def _matmul(x, w, table):
    grid = N // _RB
    return pl.pallas_call(
        _mm_body,
        grid=(grid,),
        in_specs=[
            pl.BlockSpec((_RB, D_IN), lambda i: (i, 0)),
            pl.BlockSpec((D_IN, D_OUT), lambda i: (0, 0)),
            pl.BlockSpec((_RB, 16), lambda i: (i, 0)),
        ],
        out_specs=[
            pl.BlockSpec((_RB, H), lambda i: (i, 0)),
            pl.BlockSpec((_RB, H), lambda i: (i, 0)),
            pl.BlockSpec((_RB, 1), lambda i: (i, 0)),
        ],
        out_shape=[
            jax.ShapeDtypeStruct((N, H), jnp.float32),
            jax.ShapeDtypeStruct((N, H), jnp.float32),
            jax.ShapeDtypeStruct((N, 1), jnp.float32),
        ],
    )(x, w, table)


def _scatter_body(y0_hbm, y1_hbm, row_hbm, col_hbm, zeros_hbm,
                  o0_hbm, o1_hbm, cidx_v, ridx_v, rows_v, sem, acc_sh):
    c = lax.axis_index("c")
    s = lax.axis_index("s")
    pltpu.sync_copy(zeros_hbm.at[pl.ds(s * STRIPE, STRIPE)],
                    acc_sh.at[pl.ds(s * STRIPE, STRIPE)])
    plsc.subcore_barrier()

    def run(y_hbm):
        def chunk(g, carry):
            e0 = s * EPT + g * CH
            pltpu.sync_copy(col_hbm.at[pl.ds(e0, CH)], cidx_v)
            pltpu.sync_copy(row_hbm.at[pl.ds(e0, CH)], ridx_v)
            pltpu.async_copy(y_hbm.at[cidx_v], rows_v, sem).wait()
            pltpu.sync_copy(rows_v, acc_sh.at[ridx_v], add=True)
            return carry

        lax.fori_loop(0, NCHUNK, chunk, 0)

    @pl.when(c == 0)
    def _():
        run(y0_hbm)

    @pl.when(c == 1)
    def _():
        run(y1_hbm)

    plsc.subcore_barrier()

    @pl.when(c == 0)
    def _():
        pltpu.sync_copy(acc_sh.at[pl.ds(s * STRIPE, STRIPE)],
                        o0_hbm.at[pl.ds(s * STRIPE, STRIPE)])

    @pl.when(c == 1)
    def _():
        pltpu.sync_copy(acc_sh.at[pl.ds(s * STRIPE, STRIPE)],
                        o1_hbm.at[pl.ds(s * STRIPE, STRIPE)])


_scatter = pl.kernel(
    _scatter_body,
    out_type=[
        jax.ShapeDtypeStruct((NP, H), jnp.float32),
        jax.ShapeDtypeStruct((NP, H), jnp.float32),
    ],
    mesh=_MESH,
    scratch_types=[
        pltpu.VMEM((CH,), jnp.int32),
        pltpu.VMEM((CH,), jnp.int32),
        pltpu.VMEM((CH, H), jnp.float32),
        pltpu.SemaphoreType.DMA,
        pltpu.VMEM_SHARED((NP, H), jnp.float32),
    ],
)


def _scale_body(o0_ref, o1_ref, dis_ref, out_ref):
    dis = dis_ref[...]
    out_ref[:, :H] = o0_ref[...] * dis
    out_ref[:, H:] = o1_ref[...] * dis


def _scale(o0, o1, dis):
    grid = N // _RB
    return pl.pallas_call(
        _scale_body,
        grid=(grid,),
        in_specs=[
            pl.BlockSpec((_RB, H), lambda i: (i, 0)),
            pl.BlockSpec((_RB, H), lambda i: (i, 0)),
            pl.BlockSpec((_RB, 1), lambda i: (i, 0)),
        ],
        out_specs=pl.BlockSpec((_RB, D_OUT), lambda i: (i, 0)),
        out_shape=jax.ShapeDtypeStruct((N, D_OUT), jnp.float32),
    )(o0, o1, dis)


def kernel(x, edge_index, W):
    row = edge_index[0].astype(jnp.int32)
    col = edge_index[1].astype(jnp.int32)
    ones16 = jnp.ones((CH, 16), jnp.float32)
    zeros16 = jnp.zeros((NP, 16), jnp.float32)
    zerosH = jnp.zeros((NP, H), jnp.float32)
    table = _hist(row, ones16, zeros16)
    y0, y1, dis = _matmul(x, W, table)
    o0, o1 = _scatter(y0, y1, row, col, zerosH)
    return _scale(o0, o1, dis)


# speedup vs baseline: 7.5239x; 7.5239x over previous
import functools

import jax
import jax.numpy as jnp
from jax import lax
from jax.experimental import pallas as pl
from jax.experimental.pallas import tpu as pltpu
from jax.experimental.pallas import tpu_sc as plsc

N = 10000
E = 160000
D_IN = 256
D_OUT = 256
H = 128
NP = 10240
NS = 16
STRIPE = NP // NS
CH = 80
EPT = E // NS
NCHUNK = EPT // CH

_MESH = plsc.VectorSubcoreMesh(core_axis_name="c", subcore_axis_name="s")


def _hist_body(row_hbm, ones_hbm, zeros_hbm, table_hbm, ridx_v, ones_v, table_sh):
    c = lax.axis_index("c")
    s = lax.axis_index("s")

    @pl.when(c == 0)
    def _():
        pltpu.sync_copy(zeros_hbm.at[pl.ds(s * STRIPE, STRIPE)],
                        table_sh.at[pl.ds(s * STRIPE, STRIPE)])
        pltpu.sync_copy(ones_hbm, ones_v)
        plsc.subcore_barrier()

        def chunk(g, carry):
            e0 = s * EPT + g * CH
            pltpu.sync_copy(row_hbm.at[pl.ds(e0, CH)], ridx_v)
            pltpu.sync_copy(ones_v, table_sh.at[ridx_v], add=True)
            return carry

        lax.fori_loop(0, NCHUNK, chunk, 0)
        plsc.subcore_barrier()
        pltpu.sync_copy(table_sh.at[pl.ds(s * STRIPE, STRIPE)],
                        table_hbm.at[pl.ds(s * STRIPE, STRIPE)])


_hist = pl.kernel(
    _hist_body,
    out_type=jax.ShapeDtypeStruct((NP,), jnp.float32),
    mesh=_MESH,
    scratch_types=[
        pltpu.VMEM((CH,), jnp.int32),
        pltpu.VMEM((CH,), jnp.float32),
        pltpu.VMEM_SHARED((NP,), jnp.float32),
    ],
)


def _mm_body(x_ref, w_ref, deg_ref, y0_ref, y1_ref, dis_ref):
    deg = deg_ref[...]
    dis = jnp.where(deg > 0.0, lax.rsqrt(deg), 0.0)
    y = jnp.dot(x_ref[...], w_ref[...], preferred_element_type=jnp.float32) * dis
    y0_ref[...] = y[:, :H]
    y1_ref[...] = y[:, H:]
    dis_ref[...] = dis


_RB = 1000


def _matmul(x, w, deg):
    grid = N // _RB
    return pl.pallas_call(
        _mm_body,
        grid=(grid,),
        in_specs=[
            pl.BlockSpec((_RB, D_IN), lambda i: (i, 0)),
            pl.BlockSpec((D_IN, D_OUT), lambda i: (0, 0)),
            pl.BlockSpec((_RB, 1), lambda i: (i, 0)),
        ],
        out_specs=[
            pl.BlockSpec((_RB, H), lambda i: (i, 0)),
            pl.BlockSpec((_RB, H), lambda i: (i, 0)),
            pl.BlockSpec((_RB, 1), lambda i: (i, 0)),
        ],
        out_shape=[
            jax.ShapeDtypeStruct((N, H), jnp.float32),
            jax.ShapeDtypeStruct((N, H), jnp.float32),
            jax.ShapeDtypeStruct((N, 1), jnp.float32),
        ],
    )(x, w, deg)


def _scatter_body(y0_hbm, y1_hbm, row_hbm, col_hbm, zeros_hbm,
                  o0_hbm, o1_hbm, cidx_v, ridx_v, rows_v, sem, acc_sh):
    c = lax.axis_index("c")
    s = lax.axis_index("s")
    pltpu.sync_copy(zeros_hbm.at[pl.ds(s * STRIPE, STRIPE)],
                    acc_sh.at[pl.ds(s * STRIPE, STRIPE)])
    plsc.subcore_barrier()

    def run(y_hbm):
        def chunk(g, carry):
            e0 = s * EPT + g * CH
            pltpu.sync_copy(col_hbm.at[pl.ds(e0, CH)], cidx_v)
            pltpu.sync_copy(row_hbm.at[pl.ds(e0, CH)], ridx_v)
            pltpu.async_copy(y_hbm.at[cidx_v], rows_v, sem).wait()
            pltpu.sync_copy(rows_v, acc_sh.at[ridx_v], add=True)
            return carry

        lax.fori_loop(0, NCHUNK, chunk, 0)

    @pl.when(c == 0)
    def _():
        run(y0_hbm)

    @pl.when(c == 1)
    def _():
        run(y1_hbm)

    plsc.subcore_barrier()

    @pl.when(c == 0)
    def _():
        pltpu.sync_copy(acc_sh.at[pl.ds(s * STRIPE, STRIPE)],
                        o0_hbm.at[pl.ds(s * STRIPE, STRIPE)])

    @pl.when(c == 1)
    def _():
        pltpu.sync_copy(acc_sh.at[pl.ds(s * STRIPE, STRIPE)],
                        o1_hbm.at[pl.ds(s * STRIPE, STRIPE)])


_scatter = pl.kernel(
    _scatter_body,
    out_type=[
        jax.ShapeDtypeStruct((NP, H), jnp.float32),
        jax.ShapeDtypeStruct((NP, H), jnp.float32),
    ],
    mesh=_MESH,
    scratch_types=[
        pltpu.VMEM((CH,), jnp.int32),
        pltpu.VMEM((CH,), jnp.int32),
        pltpu.VMEM((CH, H), jnp.float32),
        pltpu.SemaphoreType.DMA,
        pltpu.VMEM_SHARED((NP, H), jnp.float32),
    ],
)


def _scale_body(o0_ref, o1_ref, dis_ref, out_ref):
    dis = dis_ref[...]
    out_ref[:, :H] = o0_ref[...] * dis
    out_ref[:, H:] = o1_ref[...] * dis


def _scale(o0, o1, dis):
    grid = N // _RB
    return pl.pallas_call(
        _scale_body,
        grid=(grid,),
        in_specs=[
            pl.BlockSpec((_RB, H), lambda i: (i, 0)),
            pl.BlockSpec((_RB, H), lambda i: (i, 0)),
            pl.BlockSpec((_RB, 1), lambda i: (i, 0)),
        ],
        out_specs=pl.BlockSpec((_RB, D_OUT), lambda i: (i, 0)),
        out_shape=jax.ShapeDtypeStruct((N, D_OUT), jnp.float32),
    )(o0, o1, dis)


def kernel(x, edge_index, W):
    row = edge_index[0].astype(jnp.int32)
    col = edge_index[1].astype(jnp.int32)
    ones1 = jnp.ones((CH,), jnp.float32)
    zeros1 = jnp.zeros((NP,), jnp.float32)
    zerosH = jnp.zeros((NP, H), jnp.float32)
    deg = _hist(row, ones1, zeros1).reshape(NP, 1)
    y0, y1, dis = _matmul(x, W, deg)
    o0, o1 = _scatter(y0, y1, row, col, zerosH)
    return _scale(o0, o1, dis)


# speedup vs baseline: 13.7603x; 1.8289x over previous
import functools

import jax
import jax.numpy as jnp
from jax import lax
from jax.experimental import pallas as pl
from jax.experimental.pallas import tpu as pltpu
from jax.experimental.pallas import tpu_sc as plsc

N = 10000
E = 160000
D_IN = 256
D_OUT = 256
H = 128
NP = 10240
NS = 16
STRIPE = NP // NS
CH = 80
EPT = E // NS
NCHUNK = EPT // CH

_MESH = plsc.VectorSubcoreMesh(core_axis_name="c", subcore_axis_name="s")


CHA = 40
EPW = E // 32
NCHA = EPW // CHA
RING_A = 8
WIN_A = 4


def _hist_body(row_hbm, ones_hbm, zeros_hbm, p0_hbm, p1_hbm,
               ridx_v, ones_v, isem, ssem, table_sh):
    c = lax.axis_index("c")
    s = lax.axis_index("s")
    wid = c * 16 + s
    base = wid * EPW

    pltpu.sync_copy(zeros_hbm.at[pl.ds(s * STRIPE, STRIPE)],
                    table_sh.at[pl.ds(s * STRIPE, STRIPE)])
    pltpu.sync_copy(ones_hbm, ones_v)
    plsc.subcore_barrier()

    def idx_load(g):
        pltpu.async_copy(row_hbm.at[pl.ds(base + g * CHA, CHA)],
                         ridx_v.at[g % RING_A], isem)

    def idx_wait(g):
        pltpu.make_async_copy(row_hbm.at[pl.ds(base + g * CHA, CHA)],
                              ridx_v.at[g % RING_A], isem).wait()

    def sc_desc(g):
        return pltpu.make_async_copy(ones_v, table_sh.at[ridx_v.at[g % RING_A]],
                                     ssem)

    for g in range(WIN_A):
        idx_load(g)

    def chunk(g, carry):
        @pl.when(g >= WIN_A)
        def _():
            sc_desc(g - WIN_A).wait()

        @pl.when(g + WIN_A < NCHA)
        def _():
            idx_load(g + WIN_A)

        idx_wait(g)
        pltpu.async_copy(ones_v, table_sh.at[ridx_v.at[g % RING_A]], ssem,
                         add=True)
        return carry

    lax.fori_loop(0, NCHA, chunk, 0)
    for j in range(WIN_A, 0, -1):
        sc_desc(NCHA - j).wait()
    plsc.subcore_barrier()

    @pl.when(c == 0)
    def _():
        pltpu.sync_copy(table_sh.at[pl.ds(s * STRIPE, STRIPE)],
                        p0_hbm.at[pl.ds(s * STRIPE, STRIPE)])

    @pl.when(c == 1)
    def _():
        pltpu.sync_copy(table_sh.at[pl.ds(s * STRIPE, STRIPE)],
                        p1_hbm.at[pl.ds(s * STRIPE, STRIPE)])


_hist = pl.kernel(
    _hist_body,
    out_type=[
        jax.ShapeDtypeStruct((NP,), jnp.float32),
        jax.ShapeDtypeStruct((NP,), jnp.float32),
    ],
    mesh=_MESH,
    scratch_types=[
        pltpu.VMEM((RING_A, CHA), jnp.int32),
        pltpu.VMEM((CHA,), jnp.float32),
        pltpu.SemaphoreType.DMA,
        pltpu.SemaphoreType.DMA,
        pltpu.VMEM_SHARED((NP,), jnp.float32),
    ],
)


def _mm_body(x_ref, w_ref, p0_ref, p1_ref, y0_ref, y1_ref, dis_ref):
    deg = p0_ref[...] + p1_ref[...]
    dis = jnp.where(deg > 0.0, lax.rsqrt(deg), 0.0)
    y = jnp.dot(x_ref[...], w_ref[...], preferred_element_type=jnp.float32) * dis
    y0_ref[...] = y[:, :H]
    y1_ref[...] = y[:, H:]
    dis_ref[...] = dis


_RB = 1000


def _matmul(x, w, p0, p1):
    grid = N // _RB
    return pl.pallas_call(
        _mm_body,
        grid=(grid,),
        in_specs=[
            pl.BlockSpec((_RB, D_IN), lambda i: (i, 0)),
            pl.BlockSpec((D_IN, D_OUT), lambda i: (0, 0)),
            pl.BlockSpec((_RB, 1), lambda i: (i, 0)),
            pl.BlockSpec((_RB, 1), lambda i: (i, 0)),
        ],
        out_specs=[
            pl.BlockSpec((_RB, H), lambda i: (i, 0)),
            pl.BlockSpec((_RB, H), lambda i: (i, 0)),
            pl.BlockSpec((_RB, 1), lambda i: (i, 0)),
        ],
        out_shape=[
            jax.ShapeDtypeStruct((N, H), jnp.float32),
            jax.ShapeDtypeStruct((N, H), jnp.float32),
            jax.ShapeDtypeStruct((N, 1), jnp.float32),
        ],
    )(x, w, p0, p1)


RING_I = 4


def _scatter_body(y0_hbm, y1_hbm, row_hbm, col_hbm, zeros_hbm,
                  o0_hbm, o1_hbm, cidx_v, ridx_v, rows_v,
                  isem, gsem, ssem, acc_sh):
    c = lax.axis_index("c")
    s = lax.axis_index("s")
    base = s * EPT
    pltpu.sync_copy(zeros_hbm.at[pl.ds(s * STRIPE, STRIPE)],
                    acc_sh.at[pl.ds(s * STRIPE, STRIPE)])
    plsc.subcore_barrier()

    def idx_load(g):
        e0 = base + g * CH
        pltpu.async_copy(col_hbm.at[pl.ds(e0, CH)], cidx_v.at[g % RING_I], isem)
        pltpu.async_copy(row_hbm.at[pl.ds(e0, CH)], ridx_v.at[g % RING_I], isem)

    def idx_wait(g):
        e0 = base + g * CH
        pltpu.make_async_copy(col_hbm.at[pl.ds(e0, CH)],
                              cidx_v.at[g % RING_I], isem).wait()
        pltpu.make_async_copy(row_hbm.at[pl.ds(e0, CH)],
                              ridx_v.at[g % RING_I], isem).wait()

    def run(y_hbm):
        def gather_desc(g):
            return pltpu.make_async_copy(y_hbm.at[cidx_v.at[g % RING_I]],
                                         rows_v.at[g % 2], gsem)

        def scatter_desc(g):
            return pltpu.make_async_copy(rows_v.at[g % 2],
                                         acc_sh.at[ridx_v.at[g % RING_I]], ssem)

        for g in range(3):
            idx_load(g)
        idx_wait(0)
        pltpu.async_copy(y_hbm.at[cidx_v.at[0]], rows_v.at[0], gsem)

        def chunk(g, carry):
            gather_desc(g).wait()

            @pl.when(g >= 1)
            def _():
                scatter_desc(g - 1).wait()

            @pl.when(g + 3 < NCHUNK)
            def _():
                idx_load(g + 3)

            @pl.when(g + 1 < NCHUNK)
            def _():
                idx_wait(g + 1)
                pltpu.async_copy(y_hbm.at[cidx_v.at[(g + 1) % RING_I]],
                                 rows_v.at[(g + 1) % 2], gsem)

            pltpu.async_copy(rows_v.at[g % 2],
                             acc_sh.at[ridx_v.at[g % RING_I]], ssem, add=True)
            return carry

        lax.fori_loop(0, NCHUNK, chunk, 0)
        scatter_desc(NCHUNK - 1).wait()

    @pl.when(c == 0)
    def _():
        run(y0_hbm)

    @pl.when(c == 1)
    def _():
        run(y1_hbm)

    plsc.subcore_barrier()

    @pl.when(c == 0)
    def _():
        pltpu.sync_copy(acc_sh.at[pl.ds(s * STRIPE, STRIPE)],
                        o0_hbm.at[pl.ds(s * STRIPE, STRIPE)])

    @pl.when(c == 1)
    def _():
        pltpu.sync_copy(acc_sh.at[pl.ds(s * STRIPE, STRIPE)],
                        o1_hbm.at[pl.ds(s * STRIPE, STRIPE)])


_scatter = pl.kernel(
    _scatter_body,
    out_type=[
        jax.ShapeDtypeStruct((NP, H), jnp.float32),
        jax.ShapeDtypeStruct((NP, H), jnp.float32),
    ],
    mesh=_MESH,
    scratch_types=[
        pltpu.VMEM((RING_I, CH), jnp.int32),
        pltpu.VMEM((RING_I, CH), jnp.int32),
        pltpu.VMEM((2, CH, H), jnp.float32),
        pltpu.SemaphoreType.DMA,
        pltpu.SemaphoreType.DMA,
        pltpu.SemaphoreType.DMA,
        pltpu.VMEM_SHARED((NP, H), jnp.float32),
    ],
)


def _scale_body(o0_ref, o1_ref, dis_ref, out_ref):
    dis = dis_ref[...]
    out_ref[:, :H] = o0_ref[...] * dis
    out_ref[:, H:] = o1_ref[...] * dis


def _scale(o0, o1, dis):
    grid = N // _RB
    return pl.pallas_call(
        _scale_body,
        grid=(grid,),
        in_specs=[
            pl.BlockSpec((_RB, H), lambda i: (i, 0)),
            pl.BlockSpec((_RB, H), lambda i: (i, 0)),
            pl.BlockSpec((_RB, 1), lambda i: (i, 0)),
        ],
        out_specs=pl.BlockSpec((_RB, D_OUT), lambda i: (i, 0)),
        out_shape=jax.ShapeDtypeStruct((N, D_OUT), jnp.float32),
    )(o0, o1, dis)


def kernel(x, edge_index, W):
    row = edge_index[0].astype(jnp.int32)
    col = edge_index[1].astype(jnp.int32)
    ones1 = jnp.ones((CHA,), jnp.float32)
    zeros1 = jnp.zeros((NP,), jnp.float32)
    zerosH = jnp.zeros((NP, H), jnp.float32)
    p0, p1 = _hist(row, ones1, zeros1)
    y0, y1, dis = _matmul(x, W, p0.reshape(NP, 1), p1.reshape(NP, 1))
    o0, o1 = _scatter(y0, y1, row, col, zerosH)
    return _scale(o0, o1, dis)


# speedup vs baseline: 17.7136x; 1.2873x over previous
import functools

import jax
import jax.numpy as jnp
from jax import lax
from jax.experimental import pallas as pl
from jax.experimental.pallas import tpu as pltpu
from jax.experimental.pallas import tpu_sc as plsc

N = 10000
E = 160000
D_IN = 256
D_OUT = 256
H = 128
NP = 10240
NS = 16
STRIPE = NP // NS
CH = 80
EPT = E // NS
NCHUNK = EPT // CH

_MESH = plsc.VectorSubcoreMesh(core_axis_name="c", subcore_axis_name="s")


CHA = 40
EPW = E // 32
NCHA = EPW // CHA
RING_A = 8
WIN_A = 4


def _hist_body(row_hbm, ones_hbm, zeros_hbm, p0_hbm, p1_hbm,
               ridx_v, ones_v, isem, ssem, table_sh):
    c = lax.axis_index("c")
    s = lax.axis_index("s")
    wid = c * 16 + s
    base = wid * EPW

    pltpu.sync_copy(zeros_hbm.at[pl.ds(s * STRIPE, STRIPE)],
                    table_sh.at[pl.ds(s * STRIPE, STRIPE)])
    pltpu.sync_copy(ones_hbm, ones_v)
    plsc.subcore_barrier()

    def idx_load(g):
        pltpu.async_copy(row_hbm.at[pl.ds(base + g * CHA, CHA)],
                         ridx_v.at[g % RING_A], isem)

    def idx_wait(g):
        pltpu.make_async_copy(row_hbm.at[pl.ds(base + g * CHA, CHA)],
                              ridx_v.at[g % RING_A], isem).wait()

    def sc_desc(g):
        return pltpu.make_async_copy(ones_v, table_sh.at[ridx_v.at[g % RING_A]],
                                     ssem)

    for g in range(WIN_A):
        idx_load(g)

    def chunk(g, carry):
        @pl.when(g >= WIN_A)
        def _():
            sc_desc(g - WIN_A).wait()

        @pl.when(g + WIN_A < NCHA)
        def _():
            idx_load(g + WIN_A)

        idx_wait(g)
        pltpu.async_copy(ones_v, table_sh.at[ridx_v.at[g % RING_A]], ssem,
                         add=True)
        return carry

    lax.fori_loop(0, NCHA, chunk, 0)
    for j in range(WIN_A, 0, -1):
        sc_desc(NCHA - j).wait()
    plsc.subcore_barrier()

    @pl.when(c == 0)
    def _():
        pltpu.sync_copy(table_sh.at[pl.ds(s * STRIPE, STRIPE)],
                        p0_hbm.at[pl.ds(s * STRIPE, STRIPE)])

    @pl.when(c == 1)
    def _():
        pltpu.sync_copy(table_sh.at[pl.ds(s * STRIPE, STRIPE)],
                        p1_hbm.at[pl.ds(s * STRIPE, STRIPE)])


_hist = pl.kernel(
    _hist_body,
    out_type=[
        jax.ShapeDtypeStruct((NP,), jnp.float32),
        jax.ShapeDtypeStruct((NP,), jnp.float32),
    ],
    mesh=_MESH,
    scratch_types=[
        pltpu.VMEM((RING_A, CHA), jnp.int32),
        pltpu.VMEM((CHA,), jnp.float32),
        pltpu.SemaphoreType.DMA,
        pltpu.SemaphoreType.DMA,
        pltpu.VMEM_SHARED((NP,), jnp.float32),
    ],
)


def _mm_body(x_ref, w_ref, p0_ref, p1_ref, y0_ref, y1_ref, dis_ref):
    deg = p0_ref[...] + p1_ref[...]
    dis = jnp.where(deg > 0.0, lax.rsqrt(deg), 0.0)
    y = jnp.dot(x_ref[...], w_ref[...], preferred_element_type=jnp.float32) * dis
    y0_ref[...] = y[:, :H]
    y1_ref[...] = y[:, H:]
    dis_ref[...] = dis


_RB = 1000


def _matmul(x, w, p0, p1):
    grid = N // _RB
    return pl.pallas_call(
        _mm_body,
        grid=(grid,),
        in_specs=[
            pl.BlockSpec((_RB, D_IN), lambda i: (i, 0)),
            pl.BlockSpec((D_IN, D_OUT), lambda i: (0, 0)),
            pl.BlockSpec((_RB, 1), lambda i: (i, 0)),
            pl.BlockSpec((_RB, 1), lambda i: (i, 0)),
        ],
        out_specs=[
            pl.BlockSpec((_RB, H), lambda i: (i, 0)),
            pl.BlockSpec((_RB, H), lambda i: (i, 0)),
            pl.BlockSpec((_RB, 1), lambda i: (i, 0)),
        ],
        out_shape=[
            jax.ShapeDtypeStruct((N, H), jnp.float32),
            jax.ShapeDtypeStruct((N, H), jnp.float32),
            jax.ShapeDtypeStruct((N, 1), jnp.float32),
        ],
    )(x, w, p0, p1)


RING_I = 6
RING_R = 4


def _scatter_body(y0_hbm, y1_hbm, row_hbm, col_hbm, zeros_hbm,
                  o0_hbm, o1_hbm, cidx_v, ridx_v, rows_v,
                  isem, gsem, ssem, acc_sh):
    c = lax.axis_index("c")
    s = lax.axis_index("s")
    base = s * EPT
    pltpu.sync_copy(zeros_hbm.at[pl.ds(s * STRIPE, STRIPE)],
                    acc_sh.at[pl.ds(s * STRIPE, STRIPE)])
    plsc.subcore_barrier()

    def idx_load(g):
        e0 = base + g * CH
        pltpu.async_copy(col_hbm.at[pl.ds(e0, CH)], cidx_v.at[g % RING_I], isem)
        pltpu.async_copy(row_hbm.at[pl.ds(e0, CH)], ridx_v.at[g % RING_I], isem)

    def idx_wait(g):
        e0 = base + g * CH
        pltpu.make_async_copy(col_hbm.at[pl.ds(e0, CH)],
                              cidx_v.at[g % RING_I], isem).wait()
        pltpu.make_async_copy(row_hbm.at[pl.ds(e0, CH)],
                              ridx_v.at[g % RING_I], isem).wait()

    def run(y_hbm):
        def gather_go(g):
            pltpu.async_copy(y_hbm.at[cidx_v.at[g % RING_I]],
                             rows_v.at[g % RING_R], gsem)

        def gather_desc(g):
            return pltpu.make_async_copy(y_hbm.at[cidx_v.at[g % RING_I]],
                                         rows_v.at[g % RING_R], gsem)

        def scatter_desc(g):
            return pltpu.make_async_copy(rows_v.at[g % RING_R],
                                         acc_sh.at[ridx_v.at[g % RING_I]], ssem)

        for g in range(4):
            idx_load(g)
        for g in range(2):
            idx_wait(g)
            gather_go(g)

        def chunk(g, carry):
            gather_desc(g).wait()

            @pl.when(g >= 2)
            def _():
                scatter_desc(g - 2).wait()

            @pl.when(g + 4 < NCHUNK)
            def _():
                idx_load(g + 4)

            @pl.when(g + 2 < NCHUNK)
            def _():
                idx_wait(g + 2)
                gather_go(g + 2)

            pltpu.async_copy(rows_v.at[g % RING_R],
                             acc_sh.at[ridx_v.at[g % RING_I]], ssem, add=True)
            return carry

        lax.fori_loop(0, NCHUNK, chunk, 0)
        scatter_desc(NCHUNK - 2).wait()
        scatter_desc(NCHUNK - 1).wait()

    @pl.when(c == 0)
    def _():
        run(y0_hbm)

    @pl.when(c == 1)
    def _():
        run(y1_hbm)

    plsc.subcore_barrier()

    @pl.when(c == 0)
    def _():
        pltpu.sync_copy(acc_sh.at[pl.ds(s * STRIPE, STRIPE)],
                        o0_hbm.at[pl.ds(s * STRIPE, STRIPE)])

    @pl.when(c == 1)
    def _():
        pltpu.sync_copy(acc_sh.at[pl.ds(s * STRIPE, STRIPE)],
                        o1_hbm.at[pl.ds(s * STRIPE, STRIPE)])


_scatter = pl.kernel(
    _scatter_body,
    out_type=[
        jax.ShapeDtypeStruct((NP, H), jnp.float32),
        jax.ShapeDtypeStruct((NP, H), jnp.float32),
    ],
    mesh=_MESH,
    scratch_types=[
        pltpu.VMEM((RING_I, CH), jnp.int32),
        pltpu.VMEM((RING_I, CH), jnp.int32),
        pltpu.VMEM((RING_R, CH, H), jnp.float32),
        pltpu.SemaphoreType.DMA,
        pltpu.SemaphoreType.DMA,
        pltpu.SemaphoreType.DMA,
        pltpu.VMEM_SHARED((NP, H), jnp.float32),
    ],
)


def _scale_body(o0_ref, o1_ref, dis_ref, out_ref):
    dis = dis_ref[...]
    out_ref[:, :H] = o0_ref[...] * dis
    out_ref[:, H:] = o1_ref[...] * dis


def _scale(o0, o1, dis):
    grid = N // _RB
    return pl.pallas_call(
        _scale_body,
        grid=(grid,),
        in_specs=[
            pl.BlockSpec((_RB, H), lambda i: (i, 0)),
            pl.BlockSpec((_RB, H), lambda i: (i, 0)),
            pl.BlockSpec((_RB, 1), lambda i: (i, 0)),
        ],
        out_specs=pl.BlockSpec((_RB, D_OUT), lambda i: (i, 0)),
        out_shape=jax.ShapeDtypeStruct((N, D_OUT), jnp.float32),
    )(o0, o1, dis)


def kernel(x, edge_index, W):
    row = edge_index[0].astype(jnp.int32)
    col = edge_index[1].astype(jnp.int32)
    ones1 = jnp.ones((CHA,), jnp.float32)
    zeros1 = jnp.zeros((NP,), jnp.float32)
    zerosH = jnp.zeros((NP, H), jnp.float32)
    p0, p1 = _hist(row, ones1, zeros1)
    y0, y1, dis = _matmul(x, W, p0.reshape(NP, 1), p1.reshape(NP, 1))
    o0, o1 = _scatter(y0, y1, row, col, zerosH)
    return _scale(o0, o1, dis)
